# baseline (speedup 1.0000x reference)
import functools

import jax
import jax.numpy as jnp
from jax.experimental import pallas as pl
from jax.experimental.pallas import tpu as pltpu

ROWS = 16384
D = 2048
BLK = 512


def _norm_body(x_ref, wt_ref, o_ref):
    xb = x_ref[...]
    wt = wt_ref[...]
    p = jnp.dot(xb.astype(jnp.bfloat16), wt.astype(jnp.bfloat16),
                preferred_element_type=jnp.float32)
    o_ref[...] = jnp.sqrt(jnp.sum(p * p, axis=1))[None, None, :]


def kernel(x, routers, expert_idx):
    w = jax.lax.dynamic_index_in_dim(routers, expert_idx, axis=0,
                                     keepdims=False)
    x2 = x.reshape(ROWS, D)
    grid = ROWS // BLK
    out = pl.pallas_call(
        _norm_body,
        grid=(grid,),
        in_specs=[
            pl.BlockSpec((BLK, D), lambda i: (i, 0)),
            pl.BlockSpec((D, 8), lambda i: (0, 0)),
        ],
        out_specs=pl.BlockSpec((1, 1, BLK), lambda i: (i, 0, 0)),
        out_shape=jax.ShapeDtypeStruct((grid, 1, BLK), jnp.float32),
    )(x2, w.T)
    return out.reshape(4, 4096)


# speedup vs baseline: 1.1363x; 1.1363x over previous
import functools

import jax
import jax.numpy as jnp
from jax.experimental import pallas as pl
from jax.experimental.pallas import tpu as pltpu

ROWS = 16384
D = 2048
BLK = 2048


def _norm_body(x_ref, wt_ref, o_ref):
    xb = x_ref[...]
    wt = wt_ref[...]
    p = jnp.dot(xb.astype(jnp.bfloat16), wt.astype(jnp.bfloat16),
                preferred_element_type=jnp.float32)
    o_ref[...] = jnp.sqrt(jnp.sum(p * p, axis=1))[None, None, :]


def kernel(x, routers, expert_idx):
    w = jax.lax.dynamic_index_in_dim(routers, expert_idx, axis=0,
                                     keepdims=False)
    x2 = x.reshape(ROWS, D)
    grid = ROWS // BLK
    out = pl.pallas_call(
        _norm_body,
        grid=(grid,),
        in_specs=[
            pl.BlockSpec((BLK, D), lambda i: (i, 0)),
            pl.BlockSpec((D, 8), lambda i: (0, 0)),
        ],
        out_specs=pl.BlockSpec((1, 1, BLK), lambda i: (i, 0, 0)),
        out_shape=jax.ShapeDtypeStruct((grid, 1, BLK), jnp.float32),
    )(x2, w.T)
    return out.reshape(4, 4096)


# speedup vs baseline: 1.2068x; 1.0620x over previous
import jax
import jax.numpy as jnp
from jax import lax
from jax.experimental import pallas as pl
from jax.experimental.pallas import tpu as pltpu

ROWS = 16384
D = 2048
CH = 512
NCH = ROWS // CH
NBUF = 4


def _body(x_hbm, wt_ref, o_ref, xbufs, sems):
    wt = wt_ref[...].astype(jnp.bfloat16)

    def start_dma(c, slot):
        pltpu.make_async_copy(
            x_hbm.at[pl.ds(c * CH, CH)], xbufs.at[slot], sems.at[slot]
        ).start()

    def wait_dma(c, slot):
        pltpu.make_async_copy(
            x_hbm.at[pl.ds(c * CH, CH)], xbufs.at[slot], sems.at[slot]
        ).wait()

    for c in range(NBUF):
        start_dma(c, c)

    def step(i, _):
        slot = lax.rem(i, NBUF)
        wait_dma(i, slot)
        xb = xbufs[slot].astype(jnp.bfloat16)
        p = jnp.dot(xb, wt, preferred_element_type=jnp.float32)
        o_ref[0, pl.ds(i * CH, CH)] = jnp.sqrt(jnp.sum(p * p, axis=1))

        @pl.when(i + NBUF < NCH)
        def _():
            start_dma(i + NBUF, slot)

        return 0

    lax.fori_loop(0, NCH, step, 0)


def kernel(x, routers, expert_idx):
    w = lax.dynamic_index_in_dim(routers, expert_idx, axis=0,
                                 keepdims=False)
    x2 = x.reshape(ROWS, D)
    out = pl.pallas_call(
        _body,
        in_specs=[
            pl.BlockSpec(memory_space=pl.ANY),
            pl.BlockSpec(memory_space=pltpu.VMEM),
        ],
        out_specs=pl.BlockSpec(memory_space=pltpu.VMEM),
        out_shape=jax.ShapeDtypeStruct((1, ROWS), jnp.float32),
        scratch_shapes=[
            pltpu.VMEM((NBUF, CH, D), jnp.float32),
            pltpu.SemaphoreType.DMA((NBUF,)),
        ],
    )(x2, w.T)
    return out.reshape(4, 4096)
